# Initial kernel scaffold; baseline (speedup 1.0000x reference)
import jax
import jax.numpy as jnp
from jax.experimental import pallas as pl


def kernel(x, embedding):
    raise NotImplementedError("write your pallas kernel here")


# speedup vs baseline: 1.8758x; 1.8758x over previous
import functools

import jax
import jax.numpy as jnp
from jax import lax
from jax.experimental import pallas as pl
from jax.experimental.pallas import tpu as pltpu
from jax.experimental.pallas import tpu_sc as plsc

NUM_CORES = 2
NUM_SUBCORES = 16
NUM_WORKERS = NUM_CORES * NUM_SUBCORES
CHUNK = 128
NBUF = 8


def kernel(x, embedding):
    b0, b1 = x.shape
    total = b0 * b1
    features = embedding.shape[1]
    n_chunks = total // CHUNK
    per_w = n_chunks // NUM_WORKERS
    n_groups = per_w // NBUF

    idx = x.reshape(n_chunks, CHUNK).astype(jnp.int32)

    mesh = plsc.VectorSubcoreMesh(core_axis_name="c", subcore_axis_name="s")

    @functools.partial(
        pl.kernel,
        out_type=jax.ShapeDtypeStruct((n_chunks, CHUNK, features), jnp.float32),
        mesh=mesh,
        scratch_types=[
            pltpu.VMEM((per_w, CHUNK), jnp.int32),
            pltpu.VMEM((NBUF, CHUNK, features), jnp.float32),
            pltpu.SemaphoreType.DMA((NBUF,)),
            pltpu.SemaphoreType.DMA((NBUF,)),
        ],
        compiler_params=pltpu.CompilerParams(use_tc_tiling_on_sc=False),
    )
    def embed_gather(table_hbm, idx_hbm, out_hbm, idx_v, rows_v, gsem, wsem):
        w = lax.axis_index("s") * NUM_CORES + lax.axis_index("c")
        base = w * per_w
        pltpu.sync_copy(idx_hbm.at[pl.ds(base, per_w)], idx_v)

        def group(g0, carry):
            g_base = g0 * NBUF
            gathers = []
            for b in range(NBUF):
                gathers.append(
                    pltpu.async_copy(
                        table_hbm.at[idx_v.at[g_base + b]],
                        rows_v.at[b],
                        gsem.at[b],
                    )
                )
            writes = []
            for b in range(NBUF):
                gathers[b].wait()
                writes.append(
                    pltpu.async_copy(
                        rows_v.at[b],
                        out_hbm.at[base + g_base + b],
                        wsem.at[b],
                    )
                )
            for b in range(NBUF):
                writes[b].wait()
            return carry

        lax.fori_loop(0, n_groups, group, 0)

    out = embed_gather(embedding, idx)
    return out.reshape(b0, b1, features)
